# baseline (speedup 1.0000x reference)
import functools

import jax
import jax.numpy as jnp
from jax import lax
from jax.experimental import pallas as pl
from jax.experimental.pallas import tpu as pltpu
from jax.experimental.pallas import tpu_sc as plsc

K_CODES = 8192
D = 64
SPAN = 8
N_SEG = 8192
ROWS_PER_TILE = 512
N_TILES = N_SEG // ROWS_PER_TILE
K_CHUNK = 1024

N_ROWS = N_SEG * SPAN
NW = 32
PER_W = N_ROWS // NW
CHUNK = 128
N_CH = PER_W // CHUNK


def _vq_body(x3_ref, emb_ref, wb_ref, bb_ref, idx_ref, bnd_ref, loss_ref,
             acc_ref):
    i = pl.program_id(0)
    x3 = x3_ref[...]
    pooled = jnp.mean(x3, axis=1)
    x2 = jnp.sum(pooled * pooled, axis=1, keepdims=True)

    x2d = x3.reshape(ROWS_PER_TILE * SPAN, D)
    wb8 = jnp.broadcast_to(wb_ref[...], (8, D))
    logits = lax.dot_general(x2d, wb8, (((1,), (1,)), ((), ())),
                             preferred_element_type=jnp.float32)[:, 0:1]
    logits = logits + bb_ref[0, 0]
    bnd = (logits > 0.0).astype(jnp.float32).reshape(ROWS_PER_TILE, SPAN)
    bnd_ref[...] = bnd

    best_d = jnp.full((ROWS_PER_TILE, 1), jnp.inf, dtype=jnp.float32)
    best_i = jnp.zeros((ROWS_PER_TILE, 1), dtype=jnp.int32)
    for c in range(K_CODES // K_CHUNK):
        embc = emb_ref[pl.ds(c * K_CHUNK, K_CHUNK), :]
        esq = jnp.sum(embc * embc, axis=1, keepdims=True)
        mm = lax.dot_general(pooled, embc, (((1,), (1,)), ((), ())),
                             preferred_element_type=jnp.float32)
        dist = (x2 + esq.reshape(1, K_CHUNK)) - 2.0 * mm
        dmin = jnp.min(dist, axis=1, keepdims=True)
        amin = jnp.argmin(dist, axis=1).astype(jnp.int32)[:, None]
        amin = amin + c * K_CHUNK
        upd = dmin < best_d
        best_i = jnp.where(upd, amin, best_i)
        best_d = jnp.where(upd, dmin, best_d)
    idx_ref[...] = jnp.broadcast_to(best_i, (ROWS_PER_TILE, SPAN))

    dsum = jnp.sum(best_d)
    bsum = jnp.sum(bnd)

    @pl.when(i == 0)
    def _():
        acc_ref[0] = dsum
        acc_ref[1] = bsum

    @pl.when(i > 0)
    def _():
        acc_ref[0] = acc_ref[0] + dsum
        acc_ref[1] = acc_ref[1] + bsum

    @pl.when(i == N_TILES - 1)
    def _():
        e_latent = acc_ref[0] / jnp.float32(N_SEG * D)
        brate = acc_ref[1] / jnp.float32(N_SEG * SPAN)
        bl = (brate - jnp.float32(1.0 / SPAN)) ** 2
        loss_ref[...] = jnp.full((1, 1), 0.25 * e_latent + 0.01 * bl,
                                 dtype=jnp.float32)


def _vq_call(x3, emb_w, wb, bb2):
    return pl.pallas_call(
        _vq_body,
        grid=(N_TILES,),
        in_specs=[
            pl.BlockSpec((ROWS_PER_TILE, SPAN, D), lambda i: (i, 0, 0)),
            pl.BlockSpec((K_CODES, D), lambda i: (0, 0)),
            pl.BlockSpec((1, D), lambda i: (0, 0)),
            pl.BlockSpec((1, 1), lambda i: (0, 0)),
        ],
        out_specs=[
            pl.BlockSpec((ROWS_PER_TILE, SPAN), lambda i: (i, 0)),
            pl.BlockSpec((ROWS_PER_TILE, SPAN), lambda i: (i, 0)),
            pl.BlockSpec((1, 1), lambda i: (0, 0)),
        ],
        out_shape=[
            jax.ShapeDtypeStruct((N_SEG, SPAN), jnp.int32),
            jax.ShapeDtypeStruct((N_SEG, SPAN), jnp.float32),
            jax.ShapeDtypeStruct((1, 1), jnp.float32),
        ],
        scratch_shapes=[pltpu.SMEM((2,), jnp.float32)],
        compiler_params=pltpu.CompilerParams(
            dimension_semantics=("arbitrary",)),
    )(x3, emb_w, wb, bb2)


def _gather_body(emb_hbm, idx_hbm, out_hbm, idx_v, rows_v, sem):
    c = lax.axis_index("c")
    s = lax.axis_index("s")
    wid = s * 2 + c
    base = wid * PER_W
    pltpu.sync_copy(idx_hbm.at[pl.ds(base, PER_W)], idx_v)

    def body(j, carry):
        off = j * CHUNK
        pltpu.async_copy(emb_hbm.at[idx_v.at[pl.ds(off, CHUNK)]], rows_v,
                         sem).wait()
        pltpu.sync_copy(rows_v, out_hbm.at[pl.ds(base + off, CHUNK)])
        return carry

    lax.fori_loop(0, N_CH, body, 0)


def _gather_call(emb_w, idx_flat):
    mesh = plsc.VectorSubcoreMesh(core_axis_name="c", subcore_axis_name="s")
    fn = functools.partial(
        pl.kernel,
        mesh=mesh,
        out_type=jax.ShapeDtypeStruct((N_ROWS, D), jnp.float32),
        scratch_types=[
            pltpu.VMEM((PER_W,), jnp.int32),
            pltpu.VMEM((CHUNK, D), jnp.float32),
            pltpu.SemaphoreType.DMA,
        ],
        compiler_params=pltpu.CompilerParams(use_tc_tiling_on_sc=False),
    )(_gather_body)
    return fn(emb_w, idx_flat)


def kernel(x, emb_w, wb, bb):
    B, T, _ = x.shape
    x3 = x.reshape(N_SEG, SPAN, D)
    idx_a, bnd, loss = _vq_call(x3, emb_w, wb, bb.reshape(1, 1))
    quant = _gather_call(emb_w, idx_a.reshape(-1))
    quantized_out = quant.reshape(B, T, D)
    total_loss = loss[0, 0]
    indices_out = idx_a.reshape(B, T)
    boundaries = bnd.reshape(B, T)
    return quantized_out, total_loss, indices_out, boundaries


# speedup vs baseline: 1.0293x; 1.0293x over previous
import functools

import jax
import jax.numpy as jnp
from jax import lax
from jax.experimental import pallas as pl
from jax.experimental.pallas import tpu as pltpu
from jax.experimental.pallas import tpu_sc as plsc

K_CODES = 8192
D = 64
SPAN = 8
N_SEG = 8192
ROWS_PER_TILE = 512
N_TILES = N_SEG // ROWS_PER_TILE
K_CHUNK = 1024

N_ROWS = N_SEG * SPAN
NW = 32
PER_W = N_ROWS // NW
CHUNK = 128
N_CH = PER_W // CHUNK


def _vq_body(x3_ref, emb_ref, wb_ref, bb_ref, idx_ref, bnd_ref, loss_ref,
             acc_ref):
    i = pl.program_id(0)
    x3 = x3_ref[...]
    pooled = jnp.mean(x3, axis=1)
    x2 = jnp.sum(pooled * pooled, axis=1, keepdims=True)

    x2d = x3.reshape(ROWS_PER_TILE * SPAN, D)
    wb8 = jnp.broadcast_to(wb_ref[...], (8, D))
    logits = lax.dot_general(x2d, wb8, (((1,), (1,)), ((), ())),
                             preferred_element_type=jnp.float32)[:, 0:1]
    logits = logits + bb_ref[0, 0]
    bnd = (logits > 0.0).astype(jnp.float32).reshape(ROWS_PER_TILE, SPAN)
    bnd_ref[...] = bnd

    best_d = jnp.full((ROWS_PER_TILE, 1), jnp.inf, dtype=jnp.float32)
    best_i = jnp.zeros((ROWS_PER_TILE, 1), dtype=jnp.int32)
    for c in range(K_CODES // K_CHUNK):
        embc = emb_ref[pl.ds(c * K_CHUNK, K_CHUNK), :]
        esq = jnp.sum(embc * embc, axis=1, keepdims=True)
        mm = lax.dot_general(pooled, embc, (((1,), (1,)), ((), ())),
                             preferred_element_type=jnp.float32)
        dist = (x2 + esq.reshape(1, K_CHUNK)) - 2.0 * mm
        dmin = jnp.min(dist, axis=1, keepdims=True)
        amin = jnp.argmin(dist, axis=1).astype(jnp.int32)[:, None]
        amin = amin + c * K_CHUNK
        upd = dmin < best_d
        best_i = jnp.where(upd, amin, best_i)
        best_d = jnp.where(upd, dmin, best_d)
    idx_ref[...] = jnp.broadcast_to(best_i, (ROWS_PER_TILE, SPAN))

    dsum = jnp.sum(best_d)
    bsum = jnp.sum(bnd)

    @pl.when(i == 0)
    def _():
        acc_ref[0] = dsum
        acc_ref[1] = bsum

    @pl.when(i > 0)
    def _():
        acc_ref[0] = acc_ref[0] + dsum
        acc_ref[1] = acc_ref[1] + bsum

    @pl.when(i == N_TILES - 1)
    def _():
        e_latent = acc_ref[0] / jnp.float32(N_SEG * D)
        brate = acc_ref[1] / jnp.float32(N_SEG * SPAN)
        bl = (brate - jnp.float32(1.0 / SPAN)) ** 2
        loss_ref[...] = jnp.full((1, 1), 0.25 * e_latent + 0.01 * bl,
                                 dtype=jnp.float32)


def _vq_call(x3, emb_w, wb, bb2):
    return pl.pallas_call(
        _vq_body,
        grid=(N_TILES,),
        in_specs=[
            pl.BlockSpec((ROWS_PER_TILE, SPAN, D), lambda i: (i, 0, 0)),
            pl.BlockSpec((K_CODES, D), lambda i: (0, 0)),
            pl.BlockSpec((1, D), lambda i: (0, 0)),
            pl.BlockSpec((1, 1), lambda i: (0, 0)),
        ],
        out_specs=[
            pl.BlockSpec((ROWS_PER_TILE, SPAN), lambda i: (i, 0)),
            pl.BlockSpec((ROWS_PER_TILE, SPAN), lambda i: (i, 0)),
            pl.BlockSpec((1, 1), lambda i: (0, 0)),
        ],
        out_shape=[
            jax.ShapeDtypeStruct((N_SEG, SPAN), jnp.int32),
            jax.ShapeDtypeStruct((N_SEG, SPAN), jnp.float32),
            jax.ShapeDtypeStruct((1, 1), jnp.float32),
        ],
        scratch_shapes=[pltpu.SMEM((2,), jnp.float32)],
        compiler_params=pltpu.CompilerParams(
            dimension_semantics=("arbitrary",)),
    )(x3, emb_w, wb, bb2)


def _gather_body(emb_hbm, idx_hbm, out_hbm, idx_v, rows0_v, rows1_v,
                 sem0, sem1):
    c = lax.axis_index("c")
    s = lax.axis_index("s")
    wid = s * 2 + c
    base = wid * PER_W
    pltpu.sync_copy(idx_hbm.at[pl.ds(base, PER_W)], idx_v)

    bufs = (rows0_v, rows1_v)
    sems = (sem0, sem1)
    pltpu.async_copy(emb_hbm.at[idx_v.at[pl.ds(0, CHUNK)]], rows0_v, sem0)

    def outer(j2, carry):
        for b in range(2):
            j = j2 * 2 + b
            nxt = bufs[1 - b]
            nsem = sems[1 - b]

            @pl.when(j + 1 < N_CH)
            def _():
                off = (j + 1) * CHUNK
                pltpu.async_copy(emb_hbm.at[idx_v.at[pl.ds(off, CHUNK)]],
                                 nxt, nsem)

            pltpu.make_async_copy(emb_hbm.at[pl.ds(0, CHUNK)], bufs[b],
                                  sems[b]).wait()
            pltpu.sync_copy(bufs[b], out_hbm.at[pl.ds(base + j * CHUNK,
                                                      CHUNK)])
        return carry

    lax.fori_loop(0, N_CH // 2, outer, 0)


def _gather_call(emb_w, idx_flat):
    mesh = plsc.VectorSubcoreMesh(core_axis_name="c", subcore_axis_name="s")
    fn = functools.partial(
        pl.kernel,
        mesh=mesh,
        out_type=jax.ShapeDtypeStruct((N_ROWS, D), jnp.float32),
        scratch_types=[
            pltpu.VMEM((PER_W,), jnp.int32),
            pltpu.VMEM((CHUNK, D), jnp.float32),
            pltpu.VMEM((CHUNK, D), jnp.float32),
            pltpu.SemaphoreType.DMA,
            pltpu.SemaphoreType.DMA,
        ],
        compiler_params=pltpu.CompilerParams(use_tc_tiling_on_sc=False),
    )(_gather_body)
    return fn(emb_w, idx_flat)


def kernel(x, emb_w, wb, bb):
    B, T, _ = x.shape
    x3 = x.reshape(N_SEG, SPAN, D)
    idx_a, bnd, loss = _vq_call(x3, emb_w, wb, bb.reshape(1, 1))
    quant = _gather_call(emb_w, idx_a.reshape(-1))
    quantized_out = quant.reshape(B, T, D)
    total_loss = loss[0, 0]
    indices_out = idx_a.reshape(B, T)
    boundaries = bnd.reshape(B, T)
    return quantized_out, total_loss, indices_out, boundaries
